# baseline (speedup 1.0000x reference)
import functools

import jax
import jax.numpy as jnp
from jax import lax
from jax.experimental import pallas as pl
from jax.experimental.pallas import tpu as pltpu
from jax.experimental.pallas import tpu_sc as plsc

_N = 512
_B = 16
_NW = 32
_WPB = 2
_RPW = _N // _WPB
_CH = 32
_NCHUNK = _RPW // _CH

_ROT = ((13, 15, 26, 6), (17, 29, 16, 24))
_KS = (0x0, 0x2A, 0x1BD11BDA ^ 0x0 ^ 0x2A)


def _threefry_bits(ctr):
    x0 = jnp.zeros_like(ctr) + jnp.uint32(_KS[0])
    x1 = ctr + jnp.uint32(_KS[1])
    for i in range(5):
        for r in _ROT[i % 2]:
            x0 = x0 + x1
            x1 = (x1 << jnp.uint32(r)) | (x1 >> jnp.uint32(32 - r))
            x1 = x1 ^ x0
        x0 = x0 + jnp.uint32(_KS[(i + 1) % 3])
        x1 = x1 + jnp.uint32(_KS[(i + 2) % 3] + i + 1)
    return x0 ^ x1


def _bits_to_uniform(bits):
    fb = (bits >> jnp.uint32(9)) | jnp.uint32(0x3F800000)
    f = jax.lax.bitcast_convert_type(fb, jnp.float32) - jnp.float32(1.0)
    minv = jnp.float32(1e-6)
    span = jnp.float32((1.0 - 1e-6) - 1e-6)
    return jnp.maximum(minv, f * span + minv)


_mesh = plsc.VectorSubcoreMesh(
    core_axis_name="c", subcore_axis_name="s", num_cores=2, num_subcores=16
)


@functools.partial(
    pl.kernel,
    out_type=jax.ShapeDtypeStruct((_B, _N, _N), jnp.float32),
    mesh=_mesh,
    scratch_types=[
        pltpu.VMEM((_B + 16,), jnp.int32),
        pltpu.VMEM((_CH, _N), jnp.float32),
        pltpu.VMEM((_CH, _N), jnp.float32),
    ],
)
def _sc_sample(a_hbm, len_hbm, out_hbm, len_v, a_v, o_v):
    wid = lax.axis_index("c") * 16 + lax.axis_index("s")
    batch = wid // _WPB
    row0 = (wid % _WPB) * _RPW
    pltpu.sync_copy(len_hbm, len_v.at[pl.ds(0, _B)])
    ln_s = len_v[pl.ds(batch, 16)][0]
    lane = lax.iota(jnp.int32, 16)

    @pl.loop(0, _NCHUNK)
    def _chunk(ci):
        r_start = row0 + ci * _CH
        pltpu.sync_copy(a_hbm.at[batch, pl.ds(r_start, _CH)], a_v)

        @pl.loop(0, _CH)
        def _row(r):
            row = r_start + r
            ctr_base = batch * (_N * _N) + row * _N
            lim = ln_s & lax.shift_right_arithmetic(row - ln_s, 31)
            limv = jnp.full((16,), lim, jnp.int32)

            @pl.loop(0, _N // 16)
            def _col(j):
                col = lane + j * 16
                rv = jnp.full((16,), row, jnp.int32)
                ctr = (ctr_base + col).astype(jnp.uint32)
                u = _bits_to_uniform(_threefry_bits(ctr))
                a = a_v[r, pl.ds(j * 16, 16)]
                y = u / (u + (jnp.float32(1.0) - u) * jnp.exp(-a))
                zero = jnp.float32(0.0)
                y = jnp.where(col != rv, y, zero)
                y = jnp.where(col < limv, y, zero)
                o_v[r, pl.ds(j * 16, 16)] = y

        pltpu.sync_copy(o_v, out_hbm.at[batch, pl.ds(r_start, _CH)])


def kernel(A, lengths):
    lengths32 = lengths.astype(jnp.int32)
    return _sc_sample(A, lengths32)


# speedup vs baseline: 2.9836x; 2.9836x over previous
import functools

import jax
import jax.numpy as jnp
from jax import lax
from jax.experimental import pallas as pl
from jax.experimental.pallas import tpu as pltpu
from jax.experimental.pallas import tpu_sc as plsc

_N = 512
_B = 16

_K = 4
_ROWS = _K * _N
_NW = 32
_RPW = _ROWS // _NW
_CH = 16
_NCH = _RPW // _CH

_ROT = ((13, 15, 26, 6), (17, 29, 16, 24))
_KS = (0x0, 0x2A, 0x1BD11BDA ^ 0x0 ^ 0x2A)


def _threefry_bits(ctr):
    x0 = jnp.zeros_like(ctr) + jnp.uint32(_KS[0])
    x1 = ctr + jnp.uint32(_KS[1])
    for i in range(5):
        for r in _ROT[i % 2]:
            x0 = x0 + x1
            x1 = (x1 << jnp.uint32(r)) | (x1 >> jnp.uint32(32 - r))
            x1 = x1 ^ x0
        x0 = x0 + jnp.uint32(_KS[(i + 1) % 3])
        x1 = x1 + jnp.uint32(_KS[(i + 2) % 3] + i + 1)
    return x0 ^ x1


def _bits_to_uniform(bits):
    fb = (bits >> jnp.uint32(9)) | jnp.uint32(0x3F800000)
    f = jax.lax.bitcast_convert_type(fb, jnp.float32) - jnp.float32(1.0)
    minv = jnp.float32(1e-6)
    span = jnp.float32((1.0 - 1e-6) - 1e-6)
    return jnp.maximum(minv, f * span + minv)


_mesh = plsc.VectorSubcoreMesh(
    core_axis_name="c", subcore_axis_name="s", num_cores=2, num_subcores=16
)


@functools.partial(
    pl.kernel,
    out_type=jax.ShapeDtypeStruct((_ROWS, _N), jnp.float32),
    mesh=_mesh,
    scratch_types=[
        pltpu.VMEM((_B + 16,), jnp.int32),
        pltpu.VMEM((_CH, _N), jnp.float32),
        pltpu.VMEM((_CH, _N), jnp.float32),
    ],
)
def _sc_sample(a_hbm, len_hbm, out_hbm, len_v, a_v, o_v):
    wid = lax.axis_index("c") * 16 + lax.axis_index("s")
    row0 = wid * _RPW
    pltpu.sync_copy(len_hbm, len_v.at[pl.ds(0, _B)])
    lane = lax.iota(jnp.int32, 16)

    @pl.loop(0, _NCH)
    def _chunk(ci):
        r_start = row0 + ci * _CH
        pltpu.sync_copy(a_hbm.at[pl.ds(r_start, _CH)], a_v)

        @pl.loop(0, _CH)
        def _row(r):
            grow = r_start + r
            rib = grow & (_N - 1)
            batch = lax.shift_right_logical(grow, 9)
            ln_s = len_v[pl.ds(batch, 16)][0]
            lim = ln_s & lax.shift_right_arithmetic(rib - ln_s, 31)
            limv = jnp.full((16,), lim, jnp.int32)
            rv = jnp.full((16,), rib, jnp.int32)
            ctr_base = grow * _N

            @pl.loop(0, _N // 16)
            def _col(j):
                col = lane + j * 16
                ctr = (ctr_base + col).astype(jnp.uint32)
                u = _bits_to_uniform(_threefry_bits(ctr))
                a = a_v[r, pl.ds(j * 16, 16)]
                y = u / (u + (jnp.float32(1.0) - u) * jnp.exp(-a))
                zero = jnp.float32(0.0)
                y = jnp.where(col != rv, y, zero)
                y = jnp.where(col < limv, y, zero)
                o_v[r, pl.ds(j * 16, 16)] = y

        pltpu.sync_copy(o_v, out_hbm.at[pl.ds(r_start, _CH)])


def _tc_body(len_ref, a_ref, o_ref):
    b = pl.program_id(0) + _K
    a = a_ref[0]
    rows = jax.lax.broadcasted_iota(jnp.int32, (_N, _N), 0)
    cols = jax.lax.broadcasted_iota(jnp.int32, (_N, _N), 1)
    ctr = (b * (_N * _N) + rows * _N + cols).astype(jnp.uint32)
    u = _bits_to_uniform(_threefry_bits(ctr))
    y = u / (u + (jnp.float32(1.0) - u) * jnp.exp(-a))
    ln = len_ref[b]
    m = (rows < ln) & (cols < ln) & (rows != cols)
    o_ref[0] = jnp.where(m, y, jnp.float32(0.0))


def kernel(A, lengths):
    lengths32 = lengths.astype(jnp.int32)
    sc_out = _sc_sample(A.reshape(_B * _N, _N), lengths32)
    tc_out = pl.pallas_call(
        _tc_body,
        grid=(_B - _K,),
        in_specs=[
            pl.BlockSpec(memory_space=pltpu.SMEM),
            pl.BlockSpec((1, _N, _N), lambda b: (b + _K, 0, 0)),
        ],
        out_specs=pl.BlockSpec((1, _N, _N), lambda b: (b + _K, 0, 0)),
        out_shape=jax.ShapeDtypeStruct((_B, _N, _N), jnp.float32),
    )(lengths32, A)
    return lax.dynamic_update_slice(
        tc_out, sc_out.reshape(_K, _N, _N), (0, 0, 0)
    )
